# baseline (speedup 1.0000x reference)
import functools

import jax
import jax.numpy as jnp
from jax import lax
from jax.experimental import pallas as pl
from jax.experimental.pallas import tpu as pltpu
from jax.experimental.pallas import tpu_sc as plsc

TEMP_INV = 10.0

B = 16384
C0 = 64
C1 = 256
D = 32
U = 100000
NC = 2
NS = 16
NW = NC * NS
B_PER_W = B // NW
CHUNK = 128
NCHUNK = B_PER_W // CHUNK
GRP = 16


def _tc_level0_body(a0T_ref, cb0_ref, out_ref):
    g = a0T_ref[...]
    m = jnp.max(g, axis=0, keepdims=True)
    e = jnp.exp((g - m) * TEMP_INV)
    s = jnp.sum(e, axis=0, keepdims=True)
    w = e / s
    out_ref[...] = lax.dot_general(
        w, cb0_ref[...], (((0,), (0,)), ((), ())),
        preferred_element_type=jnp.float32,
        precision=lax.Precision.HIGHEST)


def _tc_level0(a0T, cb0):
    blk = 2048
    grid = ((U + blk - 1) // blk,)
    return pl.pallas_call(
        _tc_level0_body,
        grid=grid,
        in_specs=[
            pl.BlockSpec((C0, blk), lambda i: (0, i)),
            pl.BlockSpec((C0, D), lambda i: (0, 0)),
        ],
        out_specs=pl.BlockSpec((blk, D), lambda i: (i, 0)),
        out_shape=jax.ShapeDtypeStruct((U, D), jnp.float32),
    )(a0T, cb0)


def _row_dma_group(tbl_hbm, idx_v, dst_v, off, sem):
    def group(g, _):
        vi = idx_v[pl.ds(off + g * GRP, GRP)]
        cps = []
        for k in range(GRP):
            cps.append(pltpu.async_copy(
                tbl_hbm.at[pl.ds(vi[k], 1)],
                dst_v.at[pl.ds(off + g * GRP + k, 1)], sem))
        for cp in cps:
            cp.wait()
        return 0

    lax.fori_loop(0, CHUNK // GRP, group, 0)


def _sc_a_body(u_hbm, a1_hbm, u0all_hbm, g1_hbm, u0g_hbm,
               uid_v, g1_v, u0_v, sem_i, sem_g, sem_r, sem_w):
    wid = lax.axis_index("s") * NC + lax.axis_index("c")
    tbase = wid * B_PER_W
    pltpu.async_copy(u_hbm.at[pl.ds(tbase, B_PER_W)], uid_v, sem_i).wait()
    for c in range(NCHUNK):
        off = c * CHUNK
        cpg = pltpu.async_copy(
            a1_hbm.at[uid_v.at[pl.ds(off, CHUNK)]], g1_v, sem_g)
        _row_dma_group(u0all_hbm, uid_v, u0_v, off, sem_r)
        cpg.wait()
        wb1 = pltpu.async_copy(g1_v, g1_hbm.at[pl.ds(tbase + off, CHUNK)],
                               sem_w)
        wb1.wait()
    pltpu.sync_copy(u0_v, u0g_hbm.at[pl.ds(tbase, B_PER_W)])


@functools.cache
def _sc_a():
    return pl.kernel(
        _sc_a_body,
        out_type=[
            jax.ShapeDtypeStruct((B, C1), jnp.float32),
            jax.ShapeDtypeStruct((B, D), jnp.float32),
        ],
        mesh=plsc.VectorSubcoreMesh(core_axis_name="c", subcore_axis_name="s"),
        scratch_types=[
            pltpu.VMEM((B_PER_W,), jnp.int32),
            pltpu.VMEM((CHUNK, C1), jnp.float32),
            pltpu.VMEM((B_PER_W, D), jnp.float32),
            pltpu.SemaphoreType.DMA,
            pltpu.SemaphoreType.DMA,
            pltpu.SemaphoreType.DMA,
            pltpu.SemaphoreType.DMA,
        ],
        compiler_params=pltpu.CompilerParams(use_tc_tiling_on_sc=True),
    )


def _sc_b_body(i_hbm, it_hbm, vg_hbm, iid_v, v_v, sem_i, sem_r):
    wid = lax.axis_index("s") * NC + lax.axis_index("c")
    tbase = wid * B_PER_W
    pltpu.async_copy(i_hbm.at[pl.ds(tbase, B_PER_W)], iid_v, sem_i).wait()
    for c in range(NCHUNK):
        _row_dma_group(it_hbm, iid_v, v_v, c * CHUNK, sem_r)
    pltpu.sync_copy(v_v, vg_hbm.at[pl.ds(tbase, B_PER_W)])


@functools.cache
def _sc_b():
    return pl.kernel(
        _sc_b_body,
        out_type=[jax.ShapeDtypeStruct((B, 4 * D), jnp.float32)],
        mesh=plsc.VectorSubcoreMesh(core_axis_name="c", subcore_axis_name="s"),
        scratch_types=[
            pltpu.VMEM((B_PER_W,), jnp.int32),
            pltpu.VMEM((B_PER_W, 4 * D), jnp.float32),
            pltpu.SemaphoreType.DMA,
            pltpu.SemaphoreType.DMA,
        ],
        compiler_params=pltpu.CompilerParams(use_tc_tiling_on_sc=True),
    )


def _tc_score_body(g1_ref, u0g_ref, vg4_ref, i_ref, cb1_ref, out_ref):
    blk = g1_ref.shape[0]
    g1 = g1_ref[...]
    m1 = jnp.max(g1, axis=-1, keepdims=True)
    e1 = jnp.exp((g1 - m1) * TEMP_INV)
    s1 = jnp.sum(e1, axis=-1, keepdims=True)
    u1 = jnp.dot(e1, cb1_ref[...], preferred_element_type=jnp.float32,
                 precision=lax.Precision.HIGHEST) / s1
    lane = lax.broadcasted_iota(jnp.int32, (blk, 4 * D), 1)
    mask = (lane >> 5) == (i_ref[...] & 3)
    vsel = jnp.where(mask, vg4_ref[...], 0.0)
    row = lax.broadcasted_iota(jnp.int32, (4 * D, D), 0)
    col = lax.broadcasted_iota(jnp.int32, (4 * D, D), 1)
    sel = ((row & (D - 1)) == col).astype(jnp.float32)
    v = jnp.dot(vsel, sel, preferred_element_type=jnp.float32,
                precision=lax.Precision.HIGHEST)
    out_ref[...] = jnp.sum((u0g_ref[...] + u1) * v, axis=-1)


def _tc_score(g1, u0g, vg4, i_col, cb1):
    blk = 2048
    grid = (B // blk,)
    return pl.pallas_call(
        _tc_score_body,
        grid=grid,
        in_specs=[
            pl.BlockSpec((blk, C1), lambda i: (i, 0)),
            pl.BlockSpec((blk, D), lambda i: (i, 0)),
            pl.BlockSpec((blk, 4 * D), lambda i: (i, 0)),
            pl.BlockSpec((blk, 1), lambda i: (i, 0)),
            pl.BlockSpec((C1, D), lambda i: (0, 0)),
        ],
        out_specs=pl.BlockSpec((blk,), lambda i: (i,)),
        out_shape=jax.ShapeDtypeStruct((B,), jnp.float32),
    )(g1, u0g, vg4, i_col, cb1)


def kernel(X, assign0, codebook0, assign1, codebook1, item_table):
    user_ids = X[:, 0].astype(jnp.int32)
    item_ids = X[:, 1].astype(jnp.int32)
    u0all = _tc_level0(assign0.T, codebook0)
    item_p4 = item_table.reshape(item_table.shape[0] // 4, 4 * D)
    g1, u0g = _sc_a()(user_ids, assign1, u0all)
    vg4, = _sc_b()(item_ids >> 2, item_p4)
    out = _tc_score(g1, u0g, vg4, X[:, 1:2].astype(jnp.int32), codebook1)
    return out.reshape(B, 1)


# speedup vs baseline: 1.6225x; 1.6225x over previous
import functools

import jax
import jax.numpy as jnp
from jax import lax
from jax.experimental import pallas as pl
from jax.experimental.pallas import tpu as pltpu
from jax.experimental.pallas import tpu_sc as plsc

TEMP_INV = 10.0

B = 16384
C0 = 64
C1 = 256
D = 32
U = 100000
NC = 2
NS = 16
NW = NC * NS
B_PER_W = B // NW
CHUNK = 128
NCHUNK = B_PER_W // CHUNK
GRP = 16


def _tc_level0_body(a0T_ref, cb0_ref, out_ref):
    g = a0T_ref[...]
    m = jnp.max(g, axis=0, keepdims=True)
    e = jnp.exp((g - m) * TEMP_INV)
    s = jnp.sum(e, axis=0, keepdims=True)
    w = e / s
    out_ref[...] = lax.dot_general(
        w, cb0_ref[...], (((0,), (0,)), ((), ())),
        preferred_element_type=jnp.float32,
        precision=lax.Precision.HIGHEST)


def _tc_level0(a0T, cb0):
    blk = 2048
    grid = ((U + blk - 1) // blk,)
    return pl.pallas_call(
        _tc_level0_body,
        grid=grid,
        in_specs=[
            pl.BlockSpec((C0, blk), lambda i: (0, i)),
            pl.BlockSpec((C0, D), lambda i: (0, 0)),
        ],
        out_specs=pl.BlockSpec((blk, D), lambda i: (i, 0)),
        out_shape=jax.ShapeDtypeStruct((U, D), jnp.float32),
    )(a0T, cb0)


def _row_dma_group(tbl_hbm, idx_v, dst_v, off, sem):
    def group(g, _):
        vi = idx_v[pl.ds(off + g * GRP, GRP)]
        cps = []
        for k in range(GRP):
            cps.append(pltpu.async_copy(
                tbl_hbm.at[pl.ds(vi[k], 1)],
                dst_v.at[pl.ds(off + g * GRP + k, 1)], sem))
        for cp in cps:
            cp.wait()
        return 0

    lax.fori_loop(0, CHUNK // GRP, group, 0)


def _sc_a_body(u_hbm, a1_hbm, u0all_hbm, g1_hbm, u0g_hbm,
               uid_v, g1_v, u0_v, sem_i, sem_g, sem_r, sem_w):
    wid = lax.axis_index("s") * NC + lax.axis_index("c")
    tbase = wid * B_PER_W
    pltpu.async_copy(u_hbm.at[pl.ds(tbase, B_PER_W)], uid_v, sem_i).wait()
    for c in range(NCHUNK):
        off = c * CHUNK
        cpg = pltpu.async_copy(
            a1_hbm.at[uid_v.at[pl.ds(off, CHUNK)]], g1_v, sem_g)
        _row_dma_group(u0all_hbm, uid_v, u0_v, off, sem_r)
        cpg.wait()
        wb1 = pltpu.async_copy(g1_v, g1_hbm.at[pl.ds(tbase + off, CHUNK)],
                               sem_w)
        wb1.wait()
    pltpu.sync_copy(u0_v, u0g_hbm.at[pl.ds(tbase, B_PER_W)])


@functools.cache
def _sc_a():
    return pl.kernel(
        _sc_a_body,
        out_type=[
            jax.ShapeDtypeStruct((B, C1), jnp.float32),
            jax.ShapeDtypeStruct((B, D), jnp.float32),
        ],
        mesh=plsc.VectorSubcoreMesh(core_axis_name="c", subcore_axis_name="s"),
        scratch_types=[
            pltpu.VMEM((B_PER_W,), jnp.int32),
            pltpu.VMEM((CHUNK, C1), jnp.float32),
            pltpu.VMEM((B_PER_W, D), jnp.float32),
            pltpu.SemaphoreType.DMA,
            pltpu.SemaphoreType.DMA,
            pltpu.SemaphoreType.DMA,
            pltpu.SemaphoreType.DMA,
        ],
        compiler_params=pltpu.CompilerParams(use_tc_tiling_on_sc=True),
    )


def _tc_itemize_body(itT_ref, out_ref):
    out_ref[...] = jnp.transpose(itT_ref[...])


def _tc_itemize(itT):
    n = itT.shape[1]
    blk = 32768
    grid = ((n + blk - 1) // blk,)
    return pl.pallas_call(
        _tc_itemize_body,
        grid=grid,
        in_specs=[pl.BlockSpec((D, blk), lambda i: (0, i))],
        out_specs=pl.BlockSpec((blk, D), lambda i: (i, 0)),
        out_shape=jax.ShapeDtypeStruct((n, D), jnp.float32),
    )(itT)


def _sc_b_body(i_hbm, it_hbm, vg_hbm, iid_v, v_v, sem_i, sem_r):
    wid = lax.axis_index("s") * NC + lax.axis_index("c")
    tbase = wid * B_PER_W
    pltpu.async_copy(i_hbm.at[pl.ds(tbase, B_PER_W)], iid_v, sem_i).wait()
    for c in range(NCHUNK):
        _row_dma_group(it_hbm, iid_v, v_v, c * CHUNK, sem_r)
    pltpu.sync_copy(v_v, vg_hbm.at[pl.ds(tbase, B_PER_W)])


@functools.cache
def _sc_b():
    return pl.kernel(
        _sc_b_body,
        out_type=[jax.ShapeDtypeStruct((B, D), jnp.float32)],
        mesh=plsc.VectorSubcoreMesh(core_axis_name="c", subcore_axis_name="s"),
        scratch_types=[
            pltpu.VMEM((B_PER_W,), jnp.int32),
            pltpu.VMEM((B_PER_W, D), jnp.float32),
            pltpu.SemaphoreType.DMA,
            pltpu.SemaphoreType.DMA,
        ],
        compiler_params=pltpu.CompilerParams(use_tc_tiling_on_sc=True),
    )


def _tc_score_body(g1_ref, u0g_ref, vg_ref, cb1_ref, out_ref):
    g1 = g1_ref[...]
    m1 = jnp.max(g1, axis=-1, keepdims=True)
    e1 = jnp.exp((g1 - m1) * TEMP_INV)
    s1 = jnp.sum(e1, axis=-1, keepdims=True)
    u1 = jnp.dot(e1, cb1_ref[...], preferred_element_type=jnp.float32,
                 precision=lax.Precision.HIGHEST) / s1
    out_ref[...] = jnp.sum((u0g_ref[...] + u1) * vg_ref[...], axis=-1)


def _tc_score(g1, u0g, vg, cb1):
    blk = 2048
    grid = (B // blk,)
    return pl.pallas_call(
        _tc_score_body,
        grid=grid,
        in_specs=[
            pl.BlockSpec((blk, C1), lambda i: (i, 0)),
            pl.BlockSpec((blk, D), lambda i: (i, 0)),
            pl.BlockSpec((blk, D), lambda i: (i, 0)),
            pl.BlockSpec((C1, D), lambda i: (0, 0)),
        ],
        out_specs=pl.BlockSpec((blk,), lambda i: (i,)),
        out_shape=jax.ShapeDtypeStruct((B,), jnp.float32),
    )(g1, u0g, vg, cb1)


def kernel(X, assign0, codebook0, assign1, codebook1, item_table):
    user_ids = X[:, 0].astype(jnp.int32)
    item_ids = X[:, 1].astype(jnp.int32)
    u0all = _tc_level0(assign0.T, codebook0)
    item_rm = _tc_itemize(item_table.T)
    g1, u0g = _sc_a()(user_ids, assign1, u0all)
    vg, = _sc_b()(item_ids, item_rm)
    out = _tc_score(g1, u0g, vg, codebook1)
    return out.reshape(B, 1)


# speedup vs baseline: 1.6734x; 1.0314x over previous
import functools

import jax
import jax.numpy as jnp
from jax import lax
from jax.experimental import pallas as pl
from jax.experimental.pallas import tpu as pltpu
from jax.experimental.pallas import tpu_sc as plsc

TEMP_INV = 10.0

B = 16384
C0 = 64
C1 = 256
D = 32
U = 100000
NC = 2
NS = 16
NW = NC * NS
B_PER_W = B // NW
CHUNK = 128
NCHUNK = B_PER_W // CHUNK
GRP = 16


def _tc_level0_body(a0T_ref, cb0_ref, out_ref):
    g = a0T_ref[...]
    m = jnp.max(g, axis=0, keepdims=True)
    e = jnp.exp((g - m) * TEMP_INV)
    s = jnp.sum(e, axis=0, keepdims=True)
    w = e / s
    out_ref[...] = lax.dot_general(
        w, cb0_ref[...], (((0,), (0,)), ((), ())),
        preferred_element_type=jnp.float32,
        precision=lax.Precision.HIGHEST)


def _tc_level0(a0T, cb0):
    blk = 8192
    grid = ((U + blk - 1) // blk,)
    return pl.pallas_call(
        _tc_level0_body,
        grid=grid,
        in_specs=[
            pl.BlockSpec((C0, blk), lambda i: (0, i)),
            pl.BlockSpec((C0, D), lambda i: (0, 0)),
        ],
        out_specs=pl.BlockSpec((blk, D), lambda i: (i, 0)),
        out_shape=jax.ShapeDtypeStruct((U, D), jnp.float32),
    )(a0T, cb0)


def _row_dma_group(tbl_hbm, idx_v, dst_v, off, sem):
    def group(g, _):
        vi = idx_v[pl.ds(off + g * GRP, GRP)]
        cps = []
        for k in range(GRP):
            cps.append(pltpu.async_copy(
                tbl_hbm.at[pl.ds(vi[k], 1)],
                dst_v.at[pl.ds(off + g * GRP + k, 1)], sem))
        for cp in cps:
            cp.wait()
        return 0

    lax.fori_loop(0, CHUNK // GRP, group, 0)


def _sc_a_body(u_hbm, a1_hbm, u0all_hbm, g1_hbm, u0g_hbm,
               uid_v, g1_v, u0_v, sem_i, sem_g, sem_r, sem_w):
    wid = lax.axis_index("s") * NC + lax.axis_index("c")
    tbase = wid * B_PER_W
    pltpu.async_copy(u_hbm.at[pl.ds(tbase, B_PER_W)], uid_v, sem_i).wait()
    for c in range(NCHUNK):
        off = c * CHUNK
        cpg = pltpu.async_copy(
            a1_hbm.at[uid_v.at[pl.ds(off, CHUNK)]], g1_v, sem_g)
        _row_dma_group(u0all_hbm, uid_v, u0_v, off, sem_r)
        cpg.wait()
        wb1 = pltpu.async_copy(g1_v, g1_hbm.at[pl.ds(tbase + off, CHUNK)],
                               sem_w)
        wb1.wait()
    pltpu.sync_copy(u0_v, u0g_hbm.at[pl.ds(tbase, B_PER_W)])


@functools.cache
def _sc_a():
    return pl.kernel(
        _sc_a_body,
        out_type=[
            jax.ShapeDtypeStruct((B, C1), jnp.float32),
            jax.ShapeDtypeStruct((B, D), jnp.float32),
        ],
        mesh=plsc.VectorSubcoreMesh(core_axis_name="c", subcore_axis_name="s"),
        scratch_types=[
            pltpu.VMEM((B_PER_W,), jnp.int32),
            pltpu.VMEM((CHUNK, C1), jnp.float32),
            pltpu.VMEM((B_PER_W, D), jnp.float32),
            pltpu.SemaphoreType.DMA,
            pltpu.SemaphoreType.DMA,
            pltpu.SemaphoreType.DMA,
            pltpu.SemaphoreType.DMA,
        ],
        compiler_params=pltpu.CompilerParams(use_tc_tiling_on_sc=True),
    )


def _tc_itemize_body(itT_ref, out_ref):
    out_ref[...] = jnp.transpose(itT_ref[...])


def _tc_itemize(itT):
    n = itT.shape[1]
    blk = 32768
    grid = ((n + blk - 1) // blk,)
    return pl.pallas_call(
        _tc_itemize_body,
        grid=grid,
        in_specs=[pl.BlockSpec((D, blk), lambda i: (0, i))],
        out_specs=pl.BlockSpec((blk, D), lambda i: (i, 0)),
        out_shape=jax.ShapeDtypeStruct((n, D), jnp.float32),
    )(itT)


def _sc_b_body(i_hbm, it_hbm, vg_hbm, iid_v, v_v, sem_i, sem_r):
    wid = lax.axis_index("s") * NC + lax.axis_index("c")
    tbase = wid * B_PER_W
    pltpu.async_copy(i_hbm.at[pl.ds(tbase, B_PER_W)], iid_v, sem_i).wait()
    for c in range(NCHUNK):
        _row_dma_group(it_hbm, iid_v, v_v, c * CHUNK, sem_r)
    pltpu.sync_copy(v_v, vg_hbm.at[pl.ds(tbase, B_PER_W)])


@functools.cache
def _sc_b():
    return pl.kernel(
        _sc_b_body,
        out_type=[jax.ShapeDtypeStruct((B, D), jnp.float32)],
        mesh=plsc.VectorSubcoreMesh(core_axis_name="c", subcore_axis_name="s"),
        scratch_types=[
            pltpu.VMEM((B_PER_W,), jnp.int32),
            pltpu.VMEM((B_PER_W, D), jnp.float32),
            pltpu.SemaphoreType.DMA,
            pltpu.SemaphoreType.DMA,
        ],
        compiler_params=pltpu.CompilerParams(use_tc_tiling_on_sc=True),
    )


def _tc_score_body(g1_ref, u0g_ref, vg_ref, cb1_ref, out_ref):
    g1 = g1_ref[...]
    m1 = jnp.max(g1, axis=-1, keepdims=True)
    e1 = jnp.exp((g1 - m1) * TEMP_INV)
    s1 = jnp.sum(e1, axis=-1, keepdims=True)
    u1 = jnp.dot(e1, cb1_ref[...], preferred_element_type=jnp.float32,
                 precision=lax.Precision.HIGHEST) / s1
    out_ref[...] = jnp.sum((u0g_ref[...] + u1) * vg_ref[...], axis=-1)


def _tc_score(g1, u0g, vg, cb1):
    blk = 2048
    grid = (B // blk,)
    return pl.pallas_call(
        _tc_score_body,
        grid=grid,
        in_specs=[
            pl.BlockSpec((blk, C1), lambda i: (i, 0)),
            pl.BlockSpec((blk, D), lambda i: (i, 0)),
            pl.BlockSpec((blk, D), lambda i: (i, 0)),
            pl.BlockSpec((C1, D), lambda i: (0, 0)),
        ],
        out_specs=pl.BlockSpec((blk,), lambda i: (i,)),
        out_shape=jax.ShapeDtypeStruct((B,), jnp.float32),
    )(g1, u0g, vg, cb1)


def kernel(X, assign0, codebook0, assign1, codebook1, item_table):
    user_ids = X[:, 0].astype(jnp.int32)
    item_ids = X[:, 1].astype(jnp.int32)
    u0all = _tc_level0(assign0.T, codebook0)
    item_rm = _tc_itemize(item_table.T)
    g1, u0g = _sc_a()(user_ids, assign1, u0all)
    vg, = _sc_b()(item_ids, item_rm)
    out = _tc_score(g1, u0g, vg, codebook1)
    return out.reshape(B, 1)
